# baseline (speedup 1.0000x reference)
import dataclasses
import functools

import jax
import jax.numpy as jnp
from jax import lax
from jax.experimental import pallas as pl
from jax.experimental.pallas import tpu as pltpu
from jax.experimental.pallas import tpu_sc as plsc

K_TOP = 64
_LANES = 16
_SC_CORES = 2
_SC_SUBCORES = 16
_NW = _SC_CORES * _SC_SUBCORES
_CAP = 2048
_INT_MIN = -(2 ** 31)
_MANT = 0x7FFFFFFF


def _skey(y):
    return y ^ (lax.shift_right_arithmetic(y, 31) & jnp.int32(_MANT))


def _unkey(t):
    return t ^ (lax.shift_right_arithmetic(t, 31) & jnp.int32(_MANT))


def _encode_body(h_ref, w_ref, b_ref, out_ref, g_ref):
    acc = lax.dot_general(
        h_ref[...].astype(jnp.bfloat16), w_ref[...].astype(jnp.bfloat16),
        dimension_numbers=(((1,), (1,)), ((), ())),
        preferred_element_type=jnp.float32,
    )
    pre = acc + b_ref[...]
    out_ref[...] = pre
    bm, bn = pre.shape
    g = jnp.max(pre.reshape(bm, bn // 128, 128), axis=2)
    g_ref[...] = g.T


def _prune_body(g_ref, m_ref, gr_ref):
    g = g_ref[...]
    gr_ref[...] = g.T
    sk = _skey(lax.bitcast_convert_type(g, jnp.int32))

    def body(i, t):
        cand = t + jnp.left_shift(jnp.int32(1), 31 - i)
        cnt = jnp.sum((sk >= cand).astype(jnp.int32), axis=0, keepdims=True)
        return jnp.where(cnt >= K_TOP, cand, t)

    t0 = jnp.full((1, g.shape[1]), _INT_MIN, jnp.int32)
    t = lax.fori_loop(0, 32, body, t0)
    m = lax.bitcast_convert_type(_unkey(t), jnp.float32)
    m_ref[...] = jnp.broadcast_to(m.T, m_ref.shape)


def _sc_tau_body(pre_hbm, mrep_hbm, grows_hbm, tau_hbm,
                 row_a, row_b, mstage, gstage, gbuf, buf, taustage, taucur,
                 acc, nbuf_s, sem_a, sem_b):
    d = pre_hbm.shape[1]
    ngrp = d // 128
    rows_per_w = pre_hbm.shape[0] // _NW
    cid = lax.axis_index("c")
    sid = lax.axis_index("s")
    wid = sid * _SC_CORES + cid
    base = wid * rows_per_w

    pltpu.sync_copy(mrep_hbm.at[pl.ds(base * _LANES, rows_per_w * _LANES)],
                    mstage)
    pltpu.sync_copy(grows_hbm.at[pl.ds(base * ngrp, rows_per_w * ngrp)],
                    gstage)
    ii = lax.iota(jnp.int32, _LANES)

    def process_row(rbuf, r):
        mvec = mstage[pl.ds(r * _LANES, _LANES)]
        nbuf_s[0] = 0
        nbuf_s[1] = 0

        for q in range(ngrp // _LANES):
            gv = gstage[pl.ds(r * ngrp + q * _LANES, _LANES)]
            gm = gv >= mvec
            cg = lax.reduce_max(plsc.all_reduce_population_count(gm), (0,))

            @pl.when(cg > 0)
            def _():
                n1 = nbuf_s[1]
                plsc.store_compressed(gbuf.at[pl.ds(n1, _LANES)],
                                      ii + q * _LANES, mask=gm)
                nbuf_s[1] = n1 + cg

        ng = nbuf_s[1]

        @pl.loop(0, (ng + _LANES - 1) // _LANES)
        def _(qc):
            giv = gbuf[pl.ds(qc * _LANES, _LANES)]
            for lane in range(_LANES):
                @pl.when(qc * _LANES + lane < ng)
                def _():
                    gid = lax.reduce_max(
                        jnp.where(ii == lane, giv, jnp.int32(0)), (0,))
                    goff = gid * 128
                    vs = [rbuf[pl.ds(goff + u * _LANES, _LANES)]
                          for u in range(8)]
                    for h in range(2):
                        mh = jnp.maximum(
                            jnp.maximum(vs[4 * h], vs[4 * h + 1]),
                            jnp.maximum(vs[4 * h + 2], vs[4 * h + 3]))
                        hith = lax.reduce_max(
                            (mh >= mvec).astype(jnp.int32), (0,))

                        @pl.when(hith > 0)
                        def _():
                            for u in range(4 * h, 4 * h + 4):
                                mk = vs[u] >= mvec
                                cu = lax.reduce_max(
                                    plsc.all_reduce_population_count(mk),
                                    (0,))

                                @pl.when(cu > 0)
                                def _():
                                    n0 = nbuf_s[0]

                                    @pl.when(n0 <= _CAP - _LANES)
                                    def _():
                                        sk = _skey(
                                            plsc.bitcast(vs[u], jnp.int32))
                                        plsc.store_compressed(
                                            buf.at[pl.ds(n0, _LANES)],
                                            sk, mask=mk)
                                        nbuf_s[0] = n0 + cu

        nb = nbuf_s[0]
        buf[pl.ds(nb, _LANES)] = jnp.full((_LANES,), _INT_MIN, jnp.int32)
        nv = (nb + _LANES - 1) // _LANES
        mkey = lax.reduce_max(_skey(plsc.bitcast(mvec, jnp.int32)), (0,))

        def cond(state):
            bit, _, cnt_t = state
            return jnp.logical_and(bit != 0, cnt_t != K_TOP)

        def bit_body(state):
            bit, t, cnt_t = state
            cand = t + bit
            acc[...] = jnp.zeros((_LANES,), jnp.int32)

            @pl.loop(0, nv)
            def _(q):
                kv = buf[pl.ds(q * _LANES, _LANES)]
                acc[...] += (kv >= cand).astype(jnp.int32)

            cnt = lax.reduce_sum(acc[...], (0,))
            take = jnp.logical_and(cand > t, cnt >= K_TOP)
            return (lax.shift_right_logical(bit, 1),
                    jnp.where(take, cand, t),
                    jnp.where(take, cnt, cnt_t))

        _, t, _ = lax.while_loop(
            cond, bit_body, (jnp.int32(2 ** 30), mkey, nb))
        tauv = plsc.bitcast(jnp.full((_LANES,), _unkey(t), jnp.int32),
                            jnp.float32)
        taucur[...] = jnp.where(ii == (r % _LANES), tauv, taucur[...])

        @pl.when(r % _LANES == _LANES - 1)
        def _():
            taustage[pl.ds(r - (_LANES - 1), _LANES)] = taucur[...]

    pltpu.async_copy(pre_hbm.at[base], row_a, sem_a)

    @pl.loop(0, rows_per_w, step=2)
    def _(r):
        pltpu.async_copy(pre_hbm.at[base + r + 1], row_b, sem_b)
        pltpu.make_async_copy(pre_hbm.at[base + r], row_a, sem_a).wait()
        process_row(row_a, r)

        @pl.when(r + 2 < rows_per_w)
        def _():
            pltpu.async_copy(pre_hbm.at[base + r + 2], row_a, sem_a)

        pltpu.make_async_copy(pre_hbm.at[base + r + 1], row_b, sem_b).wait()
        process_row(row_b, r + 1)

    pltpu.sync_copy(taustage, tau_hbm.at[pl.ds(base, rows_per_w)])


def _fused_decode_body(pre_ref, tau_ref, wd_ref, b_ref, z_ref, hh_ref):
    x = pre_ref[...]
    z = jnp.where(x >= tau_ref[...], jnp.maximum(x, 0.0), 0.0)
    z_ref[...] = z

    @pl.when(pl.program_id(1) == 0)
    def _():
        hh_ref[...] = jnp.broadcast_to(b_ref[...], hh_ref.shape)

    hh_ref[...] += lax.dot_general(
        z.astype(jnp.bfloat16), wd_ref[...].astype(jnp.bfloat16),
        dimension_numbers=(((1,), (1,)), ((), ())),
        preferred_element_type=jnp.float32,
    )


def kernel(h, W_enc, b_enc, W_dec, b_dec):
    n, d_model = h.shape
    d_sae = W_enc.shape[0]

    bm, bn = 512, 2048
    b2 = b_enc.reshape(1, d_sae)
    pre, gt = pl.pallas_call(
        _encode_body,
        grid=(d_sae // bn, n // bm),
        in_specs=[
            pl.BlockSpec((bm, d_model), lambda c, r: (r, 0)),
            pl.BlockSpec((bn, d_model), lambda c, r: (c, 0)),
            pl.BlockSpec((1, bn), lambda c, r: (0, c)),
        ],
        out_specs=[
            pl.BlockSpec((bm, bn), lambda c, r: (r, c)),
            pl.BlockSpec((bn // 128, bm), lambda c, r: (c, r)),
        ],
        out_shape=[
            jax.ShapeDtypeStruct((n, d_sae), jnp.float32),
            jax.ShapeDtypeStruct((d_sae // 128, n), jnp.float32),
        ],
    )(h, W_enc, b2)

    bmm = 1024
    mrep, grows = pl.pallas_call(
        _prune_body,
        grid=(n // bmm,),
        in_specs=[pl.BlockSpec((d_sae // 128, bmm), lambda j: (0, j))],
        out_specs=[
            pl.BlockSpec((bmm, _LANES), lambda j: (j, 0)),
            pl.BlockSpec((bmm, d_sae // 128), lambda j: (j, 0)),
        ],
        out_shape=[
            jax.ShapeDtypeStruct((n, _LANES), jnp.float32),
            jax.ShapeDtypeStruct((n, d_sae // 128), jnp.float32),
        ],
    )(gt)

    rows_per_w = n // _NW
    mesh = plsc.VectorSubcoreMesh(core_axis_name="c", subcore_axis_name="s")
    cp = pltpu.CompilerParams()
    if "needs_layout_passes" in pltpu.CompilerParams.__dataclass_fields__:
        cp = dataclasses.replace(cp, needs_layout_passes=False)
    sc_tau = functools.partial(
        pl.kernel,
        mesh=mesh,
        compiler_params=cp,
        out_type=jax.ShapeDtypeStruct((n,), jnp.float32),
        scratch_types=[
            pltpu.VMEM((d_sae,), jnp.float32),
            pltpu.VMEM((d_sae,), jnp.float32),
            pltpu.VMEM((rows_per_w * _LANES,), jnp.float32),
            pltpu.VMEM((rows_per_w * (d_sae // 128),), jnp.float32),
            pltpu.VMEM((d_sae // 128 + _LANES,), jnp.int32),
            pltpu.VMEM((_CAP + _LANES,), jnp.int32),
            pltpu.VMEM((rows_per_w,), jnp.float32),
            pltpu.VMEM((_LANES,), jnp.float32),
            pltpu.VMEM((_LANES,), jnp.int32),
            pltpu.SMEM((8,), jnp.int32),
            pltpu.SemaphoreType.DMA,
            pltpu.SemaphoreType.DMA,
        ],
    )(_sc_tau_body)
    tau = sc_tau(pre, mrep.reshape(n * _LANES),
                 grows.reshape(n * (d_sae // 128)))

    bm2, bk = 1024, 1024
    b3 = b_dec.reshape(1, d_model)
    z, h_hat = pl.pallas_call(
        _fused_decode_body,
        grid=(n // bm2, d_sae // bk),
        in_specs=[
            pl.BlockSpec((bm2, bk), lambda r, k: (r, k)),
            pl.BlockSpec((bm2, 1), lambda r, k: (r, 0)),
            pl.BlockSpec((d_model, bk), lambda r, k: (0, k)),
            pl.BlockSpec((1, d_model), lambda r, k: (0, 0)),
        ],
        out_specs=[
            pl.BlockSpec((bm2, bk), lambda r, k: (r, k)),
            pl.BlockSpec((bm2, d_model), lambda r, k: (r, 0)),
        ],
        out_shape=[
            jax.ShapeDtypeStruct((n, d_sae), jnp.float32),
            jax.ShapeDtypeStruct((n, d_model), jnp.float32),
        ],
    )(pre, tau.reshape(n, 1), W_dec, b3)

    return (h_hat, z)


# speedup vs baseline: 1.3400x; 1.3400x over previous
import dataclasses
import functools

import jax
import jax.numpy as jnp
from jax import lax
from jax.experimental import pallas as pl
from jax.experimental.pallas import tpu as pltpu
from jax.experimental.pallas import tpu_sc as plsc

K_TOP = 64
_LANES = 16
_SC_CORES = 2
_SC_SUBCORES = 16
_NW = _SC_CORES * _SC_SUBCORES
_CAP = 2048
_INT_MIN = -(2 ** 31)
_MANT = 0x7FFFFFFF


def _skey(y):
    return y ^ (lax.shift_right_arithmetic(y, 31) & jnp.int32(_MANT))


def _unkey(t):
    return t ^ (lax.shift_right_arithmetic(t, 31) & jnp.int32(_MANT))


def _encode_body(h_ref, w_ref, b_ref, out_ref, g_ref):
    acc = lax.dot_general(
        h_ref[...].astype(jnp.bfloat16), w_ref[...].astype(jnp.bfloat16),
        dimension_numbers=(((1,), (1,)), ((), ())),
        preferred_element_type=jnp.float32,
    )
    pre = acc + b_ref[...]
    out_ref[...] = pre
    bm, bn = pre.shape
    g = jnp.max(pre.reshape(bm, bn // 128, 128), axis=2)
    g_ref[...] = g.T


def _prune_body(g_ref, m_ref):
    g = g_ref[...]
    sk = _skey(lax.bitcast_convert_type(g, jnp.int32))

    def body(i, t):
        cand = t + jnp.left_shift(jnp.int32(1), 31 - i)
        cnt = jnp.sum((sk >= cand).astype(jnp.int32), axis=0, keepdims=True)
        return jnp.where(cnt >= K_TOP, cand, t)

    t0 = jnp.full((1, g.shape[1]), _INT_MIN, jnp.int32)
    t = lax.fori_loop(0, 32, body, t0)
    m = lax.bitcast_convert_type(_unkey(t), jnp.float32)
    m_ref[...] = jnp.broadcast_to(m.T, m_ref.shape)


def _sc_tau_body(pre_hbm, mrep_hbm, tau_hbm,
                 row_a, row_b, mstage, buf, taustage, taucur, acc,
                 nbuf_s, sem_a, sem_b):
    d = pre_hbm.shape[1]
    rows_per_w = pre_hbm.shape[0] // _NW
    cid = lax.axis_index("c")
    sid = lax.axis_index("s")
    wid = sid * _SC_CORES + cid
    base = wid * rows_per_w

    pltpu.sync_copy(mrep_hbm.at[pl.ds(base * _LANES, rows_per_w * _LANES)],
                    mstage)

    def process_row(rbuf, r):
        mvec = mstage[pl.ds(r * _LANES, _LANES)]
        nbuf_s[0] = 0

        @pl.loop(0, d, step=128)
        def _(j):
            vs = [rbuf[pl.ds(j + u * _LANES, _LANES)] for u in range(8)]
            mx = vs[0]
            for u in range(1, 8):
                mx = jnp.maximum(mx, vs[u])
            hit = lax.reduce_max((mx >= mvec).astype(jnp.int32), (0,))

            @pl.when(hit > 0)
            def _():
                for u in range(8):
                    mk = vs[u] >= mvec
                    cu = lax.reduce_max(
                        plsc.all_reduce_population_count(mk), (0,))

                    @pl.when(cu > 0)
                    def _():
                        n0 = nbuf_s[0]

                        @pl.when(n0 <= _CAP - _LANES)
                        def _():
                            sk = _skey(plsc.bitcast(vs[u], jnp.int32))
                            plsc.store_compressed(
                                buf.at[pl.ds(n0, _LANES)], sk, mask=mk)
                            nbuf_s[0] = n0 + cu

        nb = nbuf_s[0]
        buf[pl.ds(nb, _LANES)] = jnp.full((_LANES,), _INT_MIN, jnp.int32)
        nv = (nb + _LANES - 1) // _LANES
        mkey = lax.reduce_max(_skey(plsc.bitcast(mvec, jnp.int32)), (0,))

        def cond(state):
            bit, _, cnt_t = state
            return jnp.logical_and(bit != 0, cnt_t != K_TOP)

        def bit_body(state):
            bit, t, cnt_t = state
            cand = t + bit
            acc[...] = jnp.zeros((_LANES,), jnp.int32)

            @pl.loop(0, nv)
            def _(q):
                kv = buf[pl.ds(q * _LANES, _LANES)]
                acc[...] += (kv >= cand).astype(jnp.int32)

            cnt = lax.reduce_sum(acc[...], (0,))
            take = jnp.logical_and(cand > t, cnt >= K_TOP)
            return (lax.shift_right_logical(bit, 1),
                    jnp.where(take, cand, t),
                    jnp.where(take, cnt, cnt_t))

        _, t, _ = lax.while_loop(
            cond, bit_body, (jnp.int32(2 ** 30), mkey, nb))
        tauv = plsc.bitcast(jnp.full((_LANES,), _unkey(t), jnp.int32),
                            jnp.float32)
        ii = lax.iota(jnp.int32, _LANES)
        taucur[...] = jnp.where(ii == (r % _LANES), tauv, taucur[...])

        @pl.when(r % _LANES == _LANES - 1)
        def _():
            taustage[pl.ds(r - (_LANES - 1), _LANES)] = taucur[...]

    pltpu.async_copy(pre_hbm.at[base], row_a, sem_a)

    @pl.loop(0, rows_per_w, step=2)
    def _(r):
        pltpu.async_copy(pre_hbm.at[base + r + 1], row_b, sem_b)
        pltpu.make_async_copy(pre_hbm.at[base + r], row_a, sem_a).wait()
        process_row(row_a, r)

        @pl.when(r + 2 < rows_per_w)
        def _():
            pltpu.async_copy(pre_hbm.at[base + r + 2], row_a, sem_a)

        pltpu.make_async_copy(pre_hbm.at[base + r + 1], row_b, sem_b).wait()
        process_row(row_b, r + 1)

    pltpu.sync_copy(taustage, tau_hbm.at[pl.ds(base, rows_per_w)])


def _fused_decode_body(pre_ref, tau_ref, wd_ref, b_ref, z_ref, hh_ref):
    x = pre_ref[...]
    z = jnp.where(x >= tau_ref[...], jnp.maximum(x, 0.0), 0.0)
    z_ref[...] = z

    @pl.when(pl.program_id(1) == 0)
    def _():
        hh_ref[...] = jnp.broadcast_to(b_ref[...], hh_ref.shape)

    hh_ref[...] += lax.dot_general(
        z.astype(jnp.bfloat16), wd_ref[...].astype(jnp.bfloat16),
        dimension_numbers=(((1,), (1,)), ((), ())),
        preferred_element_type=jnp.float32,
    )


def kernel(h, W_enc, b_enc, W_dec, b_dec):
    n, d_model = h.shape
    d_sae = W_enc.shape[0]

    bm, bn = 512, 2048
    b2 = b_enc.reshape(1, d_sae)
    pre, gt = pl.pallas_call(
        _encode_body,
        grid=(d_sae // bn, n // bm),
        in_specs=[
            pl.BlockSpec((bm, d_model), lambda c, r: (r, 0)),
            pl.BlockSpec((bn, d_model), lambda c, r: (c, 0)),
            pl.BlockSpec((1, bn), lambda c, r: (0, c)),
        ],
        out_specs=[
            pl.BlockSpec((bm, bn), lambda c, r: (r, c)),
            pl.BlockSpec((bn // 128, bm), lambda c, r: (c, r)),
        ],
        out_shape=[
            jax.ShapeDtypeStruct((n, d_sae), jnp.float32),
            jax.ShapeDtypeStruct((d_sae // 128, n), jnp.float32),
        ],
    )(h, W_enc, b2)

    bmm = 1024
    mrep = pl.pallas_call(
        _prune_body,
        grid=(n // bmm,),
        in_specs=[pl.BlockSpec((d_sae // 128, bmm), lambda j: (0, j))],
        out_specs=pl.BlockSpec((bmm, _LANES), lambda j: (j, 0)),
        out_shape=jax.ShapeDtypeStruct((n, _LANES), jnp.float32),
    )(gt)

    rows_per_w = n // _NW
    mesh = plsc.VectorSubcoreMesh(core_axis_name="c", subcore_axis_name="s")
    cp = pltpu.CompilerParams()
    if "needs_layout_passes" in pltpu.CompilerParams.__dataclass_fields__:
        cp = dataclasses.replace(cp, needs_layout_passes=False)
    sc_tau = functools.partial(
        pl.kernel,
        mesh=mesh,
        compiler_params=cp,
        out_type=jax.ShapeDtypeStruct((n,), jnp.float32),
        scratch_types=[
            pltpu.VMEM((d_sae,), jnp.float32),
            pltpu.VMEM((d_sae,), jnp.float32),
            pltpu.VMEM((rows_per_w * _LANES,), jnp.float32),
            pltpu.VMEM((_CAP + _LANES,), jnp.int32),
            pltpu.VMEM((rows_per_w,), jnp.float32),
            pltpu.VMEM((_LANES,), jnp.float32),
            pltpu.VMEM((_LANES,), jnp.int32),
            pltpu.SMEM((8,), jnp.int32),
            pltpu.SemaphoreType.DMA,
            pltpu.SemaphoreType.DMA,
        ],
    )(_sc_tau_body)
    tau = sc_tau(pre, mrep.reshape(n * _LANES))

    bm2, bk = 1024, 1024
    b3 = b_dec.reshape(1, d_model)
    z, h_hat = pl.pallas_call(
        _fused_decode_body,
        grid=(n // bm2, d_sae // bk),
        in_specs=[
            pl.BlockSpec((bm2, bk), lambda r, k: (r, k)),
            pl.BlockSpec((bm2, 1), lambda r, k: (r, 0)),
            pl.BlockSpec((d_model, bk), lambda r, k: (0, k)),
            pl.BlockSpec((1, d_model), lambda r, k: (0, 0)),
        ],
        out_specs=[
            pl.BlockSpec((bm2, bk), lambda r, k: (r, k)),
            pl.BlockSpec((bm2, d_model), lambda r, k: (r, 0)),
        ],
        out_shape=[
            jax.ShapeDtypeStruct((n, d_sae), jnp.float32),
            jax.ShapeDtypeStruct((n, d_model), jnp.float32),
        ],
    )(pre, tau.reshape(n, 1), W_dec, b3)

    return (h_hat, z)
